# Initial kernel scaffold; baseline (speedup 1.0000x reference)
import jax
import jax.numpy as jnp
from jax.experimental import pallas as pl


def kernel(target, context, target_table, context_table):
    raise NotImplementedError("write your pallas kernel here")


# speedup vs baseline: 15.7360x; 15.7360x over previous
import functools

import jax
import jax.numpy as jnp
from jax import lax
from jax.experimental import pallas as pl
from jax.experimental.pallas import tpu as pltpu
from jax.experimental.pallas import tpu_sc as plsc

E = 128
C = 5
B = 16384
NC = 2
NS = 16
NW = NC * NS
BPW = B // NW
CB = 16
NCHUNK = BPW // CB
LANES = 8


def _w2v_body(tgt_hbm, ctx_hbm, ttab_hbm, ctab_hbm, out_hbm,
              tidx_v, cidx_v, wbuf, cbuf, out_v, sem_w, sem_c):
    wid = lax.axis_index("s") * NC + lax.axis_index("c")

    pltpu.sync_copy(tgt_hbm.at[wid], tidx_v)
    pltpu.sync_copy(ctx_hbm.at[wid], cidx_v)

    def chunk_body(k, _):
        cpw = pltpu.async_copy(ttab_hbm.at[tidx_v.at[k]], wbuf, sem_w)
        cpc = pltpu.async_copy(ctab_hbm.at[cidx_v.at[k]], cbuf, sem_c)
        cpw.wait()
        cpc.wait()

        def row_body(j, _):
            w = [wbuf[j, pl.ds(16 * t, 16)] for t in range(LANES)]
            lane = lax.iota(jnp.int32, 16)
            vec = jnp.zeros((16,), jnp.float32)
            for c in range(C):
                r = j * C + c
                acc = w[0] * cbuf[r, pl.ds(0, 16)]
                for t in range(1, LANES):
                    acc = acc + w[t] * cbuf[r, pl.ds(16 * t, 16)]
                vec = jnp.where(lane == c, jnp.sum(acc), vec)
            out_v[k * CB + j, pl.ds(0, 16)] = vec
            return _

        lax.fori_loop(0, CB, row_body, None)
        return _

    lax.fori_loop(0, NCHUNK, chunk_body, None)
    pltpu.sync_copy(out_v, out_hbm.at[wid])


@jax.jit
def _w2v(tgt, ctx, ttab, ctab):
    mesh = plsc.VectorSubcoreMesh(core_axis_name="c", subcore_axis_name="s")
    f = functools.partial(
        pl.kernel,
        out_type=jax.ShapeDtypeStruct((NW, BPW, 16), jnp.float32),
        mesh=mesh,
        compiler_params=pltpu.CompilerParams(needs_layout_passes=False),
        scratch_types=[
            pltpu.VMEM((NCHUNK, CB), jnp.int32),
            pltpu.VMEM((NCHUNK, CB * C), jnp.int32),
            pltpu.VMEM((CB, E), jnp.float32),
            pltpu.VMEM((CB * C, E), jnp.float32),
            pltpu.VMEM((BPW, 16), jnp.float32),
            pltpu.SemaphoreType.DMA,
            pltpu.SemaphoreType.DMA,
        ],
    )(_w2v_body)
    return f(tgt, ctx, ttab, ctab)


def kernel(target, context, target_table, context_table):
    if target.ndim == 2:
        target = jnp.squeeze(target, axis=1)
    tgt = target.reshape(NW, NCHUNK, CB)
    ctx = context.reshape(NW, NCHUNK, CB * C)
    out = _w2v(tgt, ctx, target_table, context_table)
    return out.reshape(B, 16)[:, :C]


# speedup vs baseline: 21.4419x; 1.3626x over previous
import functools

import jax
import jax.numpy as jnp
from jax import lax
from jax.experimental import pallas as pl
from jax.experimental.pallas import tpu as pltpu
from jax.experimental.pallas import tpu_sc as plsc

E = 128
C = 5
B = 16384
NC = 2
NS = 16
NW = NC * NS
BPW = B // NW
CB = 16
NCHUNK = BPW // CB
LANES = 8


def _w2v_body(tgt_hbm, ctx_hbm, ttab_hbm, ctab_hbm, out_hbm,
              tidx_v, cidx_v, wbuf, cbuf, out_v,
              sem_w0, sem_c0, sem_w1, sem_c1):
    wid = lax.axis_index("s") * NC + lax.axis_index("c")

    pltpu.sync_copy(tgt_hbm.at[wid], tidx_v)
    pltpu.sync_copy(ctx_hbm.at[wid], cidx_v)

    sems = ((sem_w0, sem_c0), (sem_w1, sem_c1))

    def start(k, slot):
        sw, sc = sems[slot]
        pltpu.make_async_copy(ttab_hbm.at[tidx_v.at[k]], wbuf.at[slot], sw).start()
        pltpu.make_async_copy(ctab_hbm.at[cidx_v.at[k]], cbuf.at[slot], sc).start()

    def wait(slot):
        sw, sc = sems[slot]
        pltpu.make_async_copy(ttab_hbm.at[pl.ds(0, CB)], wbuf.at[slot], sw).wait()
        pltpu.make_async_copy(ctab_hbm.at[pl.ds(0, CB * C)], cbuf.at[slot], sc).wait()

    def compute(k, slot):
        def row_body(j, _):
            w = [wbuf[slot, j, pl.ds(16 * t, 16)] for t in range(LANES)]
            lane = lax.iota(jnp.int32, 16)
            vec = jnp.zeros((16,), jnp.float32)
            for c in range(C):
                r = j * C + c
                acc = w[0] * cbuf[slot, r, pl.ds(0, 16)]
                for t in range(1, LANES):
                    acc = acc + w[t] * cbuf[slot, r, pl.ds(16 * t, 16)]
                vec = jnp.where(lane == c, jnp.sum(acc), vec)
            out_v[k * CB + j, pl.ds(0, 16)] = vec
            return _

        lax.fori_loop(0, CB, row_body, None)

    start(0, 0)

    def pair_body(i, _):
        k = 2 * i
        start(k + 1, 1)
        wait(0)
        compute(k, 0)

        @pl.when(k + 2 < NCHUNK)
        def _prefetch():
            start(k + 2, 0)

        wait(1)
        compute(k + 1, 1)
        return _

    lax.fori_loop(0, NCHUNK // 2, pair_body, None)
    pltpu.sync_copy(out_v, out_hbm.at[wid])


@jax.jit
def _w2v(tgt, ctx, ttab, ctab):
    mesh = plsc.VectorSubcoreMesh(core_axis_name="c", subcore_axis_name="s")
    f = functools.partial(
        pl.kernel,
        out_type=jax.ShapeDtypeStruct((NW, BPW, 16), jnp.float32),
        mesh=mesh,
        compiler_params=pltpu.CompilerParams(needs_layout_passes=False),
        scratch_types=[
            pltpu.VMEM((NCHUNK, CB), jnp.int32),
            pltpu.VMEM((NCHUNK, CB * C), jnp.int32),
            pltpu.VMEM((2, CB, E), jnp.float32),
            pltpu.VMEM((2, CB * C, E), jnp.float32),
            pltpu.VMEM((BPW, 16), jnp.float32),
            pltpu.SemaphoreType.DMA,
            pltpu.SemaphoreType.DMA,
            pltpu.SemaphoreType.DMA,
            pltpu.SemaphoreType.DMA,
        ],
    )(_w2v_body)
    return f(tgt, ctx, ttab, ctab)


def kernel(target, context, target_table, context_table):
    if target.ndim == 2:
        target = jnp.squeeze(target, axis=1)
    tgt = target.reshape(NW, NCHUNK, CB)
    ctx = context.reshape(NW, NCHUNK, CB * C)
    out = _w2v(tgt, ctx, target_table, context_table)
    return out.reshape(B, 16)[:, :C]


# speedup vs baseline: 22.9818x; 1.0718x over previous
import functools

import jax
import jax.numpy as jnp
from jax import lax
from jax.experimental import pallas as pl
from jax.experimental.pallas import tpu as pltpu
from jax.experimental.pallas import tpu_sc as plsc

E = 128
C = 5
B = 16384
NC = 2
NS = 16
NW = NC * NS
BPW = B // NW
CB = 16
NCHUNK = BPW // CB
NBUF = 4
LANES = 8


def _w2v_body(tgt_hbm, ctx_hbm, ttab_hbm, ctab_hbm, out_hbm,
              tidx_v, cidx_v, wbuf, cbuf, out_v, *sems):
    wid = lax.axis_index("s") * NC + lax.axis_index("c")

    pltpu.sync_copy(tgt_hbm.at[pl.ds(wid * BPW, BPW)], tidx_v)
    pltpu.sync_copy(ctx_hbm.at[pl.ds(wid * BPW * C, BPW * C)], cidx_v)

    def start(k, slot):
        sw, sc = sems[2 * slot], sems[2 * slot + 1]
        pltpu.make_async_copy(
            ttab_hbm.at[tidx_v.at[pl.ds(k * CB, CB)]], wbuf.at[slot], sw
        ).start()
        pltpu.make_async_copy(
            ctab_hbm.at[cidx_v.at[pl.ds(k * CB * C, CB * C)]], cbuf.at[slot], sc
        ).start()

    def wait(slot):
        sw, sc = sems[2 * slot], sems[2 * slot + 1]
        pltpu.make_async_copy(ttab_hbm.at[pl.ds(0, CB)], wbuf.at[slot], sw).wait()
        pltpu.make_async_copy(ctab_hbm.at[pl.ds(0, CB * C)], cbuf.at[slot], sc).wait()

    lane = lax.iota(jnp.int32, 16)

    def compute(k, slot):
        def pair_body(p, _):
            vec = jnp.zeros((16,), jnp.float32)
            for half in range(2):
                j = 2 * p + half
                w = [wbuf[slot, j, pl.ds(16 * t, 16)] for t in range(LANES)]
                for c in range(C):
                    r = j * C + c
                    acc = w[0] * cbuf[slot, r, pl.ds(0, 16)]
                    for t in range(1, LANES):
                        acc = acc + w[t] * cbuf[slot, r, pl.ds(16 * t, 16)]
                    vec = jnp.where(lane == 8 * half + c, jnp.sum(acc), vec)
            out_v[k * (CB // 2) + p, pl.ds(0, 16)] = vec
            return _

        lax.fori_loop(0, CB // 2, pair_body, None)

    for s in range(NBUF - 1):
        start(s, s)

    def group_body(g, _):
        for b in range(NBUF):
            k = g * NBUF + b

            @pl.when(k + NBUF - 1 < NCHUNK)
            def _prefetch():
                start(k + NBUF - 1, (b + NBUF - 1) % NBUF)

            wait(b)
            compute(k, b)
        return _

    lax.fori_loop(0, NCHUNK // NBUF, group_body, None)
    pltpu.sync_copy(out_v, out_hbm.at[wid])


@jax.jit
def _w2v(tgt, ctx, ttab, ctab):
    mesh = plsc.VectorSubcoreMesh(core_axis_name="c", subcore_axis_name="s")
    f = functools.partial(
        pl.kernel,
        out_type=jax.ShapeDtypeStruct((NW, BPW // 2, 16), jnp.float32),
        mesh=mesh,
        compiler_params=pltpu.CompilerParams(needs_layout_passes=False),
        scratch_types=[
            pltpu.VMEM((BPW,), jnp.int32),
            pltpu.VMEM((BPW * C,), jnp.int32),
            pltpu.VMEM((NBUF, CB, E), jnp.float32),
            pltpu.VMEM((NBUF, CB * C, E), jnp.float32),
            pltpu.VMEM((BPW // 2, 16), jnp.float32),
        ] + [pltpu.SemaphoreType.DMA] * (2 * NBUF),
    )(_w2v_body)
    return f(tgt, ctx, ttab, ctab)


def kernel(target, context, target_table, context_table):
    if target.ndim == 2:
        target = jnp.squeeze(target, axis=1)
    ctx = context.reshape(B * C)
    out = _w2v(target, ctx, target_table, context_table)
    return out.reshape(B, 8)[:, :C]
